# baseline (speedup 1.0000x reference)
import functools

import jax
import jax.numpy as jnp
from jax import lax
from jax.experimental import pallas as pl
from jax.experimental.pallas import tpu as pltpu
from jax.experimental.pallas import tpu_sc as plsc

N = 10000
D = 256
DH = 128
E = 160000
G = 64
BN_EPS = 1e-5

_NC = 2
_NS = 16
_EC = E // _NS
_CH = 80
_IB = 64
_NCH = _EC // _CH
_WS = 624
_WT = N - _WS * _NS
_ZR = 16

_PREC = lax.Precision.DEFAULT
_BM = 1000


def _dot(a, b):
    return lax.dot_general(a, b, (((1,), (0,)), ((), ())),
                           preferred_element_type=jnp.float32,
                           precision=_PREC)


@jax.jit
def _sc_agg(h2, src, dst):
    mesh = plsc.VectorSubcoreMesh(core_axis_name="c", subcore_axis_name="s",
                                  num_cores=_NC, num_subcores=_NS)

    @functools.partial(
        pl.kernel,
        out_type=jax.ShapeDtypeStruct((_NC, N, DH), jnp.float32),
        mesh=mesh,
        scratch_types=[
            pltpu.VMEM_SHARED((N, DH), jnp.float32),
            pltpu.VMEM((_ZR, DH), jnp.float32),
            pltpu.VMEM((_IB, _CH), jnp.int32),
            pltpu.VMEM((_IB, _CH), jnp.int32),
            pltpu.VMEM((_CH, DH), jnp.float32),
            pltpu.VMEM((_CH, DH), jnp.float32),
            pltpu.SemaphoreType.DMA,
            pltpu.SemaphoreType.DMA,
            pltpu.SemaphoreType.DMA,
            pltpu.SemaphoreType.DMA,
        ],
    )
    def k(h2_hbm, src_hbm, dst_hbm, out_hbm, acc_sh, zbuf, src_v, dst_v,
          rows0, rows1, semg0, semg1, sems0, sems1):
        c = lax.axis_index("c")
        s = lax.axis_index("s")

        zv = jnp.zeros((16,), jnp.float32)

        @pl.loop(0, _ZR)
        def _(r):
            @pl.loop(0, DH, step=16)
            def _(cc):
                zbuf.at[pl.ds(r, 1), pl.ds(cc, 16)][...] = zv.reshape(1, 16)

        @pl.loop(0, _WS, step=_ZR)
        def _(r0):
            pltpu.sync_copy(zbuf, acc_sh.at[pl.ds(s * _WS + r0, _ZR)])

        @pl.when(s == 0)
        def _():
            pltpu.sync_copy(zbuf, acc_sh.at[pl.ds(_WS * _NS, _WT)])

        plsc.subcore_barrier()

        def start_gather(j, rows, semg):
            pltpu.async_copy(h2_hbm.at[c].at[src_v.at[j]], rows, semg)

        def wait_gather(j, rows, semg):
            pltpu.make_async_copy(h2_hbm.at[c].at[src_v.at[j]], rows,
                                  semg).wait()

        def start_scatter(j, rows, sems):
            pltpu.async_copy(rows, acc_sh.at[dst_v.at[j]], sems, add=True)

        def wait_scatter(j, rows, sems):
            pltpu.make_async_copy(rows, acc_sh.at[dst_v.at[j]], sems).wait()

        def pipeline_block(off, n):
            pltpu.sync_copy(src_hbm.at[s].at[pl.ds(off, n)],
                            src_v.at[pl.ds(0, n)])
            pltpu.sync_copy(dst_hbm.at[s].at[pl.ds(off, n)],
                            dst_v.at[pl.ds(0, n)])
            start_gather(0, rows0, semg0)
            p = (n - 1) // 2

            @pl.loop(0, p)
            def _(jj):
                base = 2 * jj
                @pl.when(jj > 0)
                def _():
                    wait_scatter(base - 1, rows1, sems1)

                start_gather(base + 1, rows1, semg1)
                wait_gather(base, rows0, semg0)
                start_scatter(base, rows0, sems0)
                wait_scatter(base, rows0, sems0)
                start_gather(base + 2, rows0, semg0)
                wait_gather(base + 1, rows1, semg1)
                start_scatter(base + 1, rows1, sems1)

            if p > 0:
                wait_scatter(2 * p - 1, rows1, sems1)
            wait_gather(2 * p, rows0, semg0)
            if n - 2 * p == 2:
                start_gather(2 * p + 1, rows1, semg1)
            start_scatter(2 * p, rows0, sems0)
            wait_scatter(2 * p, rows0, sems0)
            if n - 2 * p == 2:
                wait_gather(2 * p + 1, rows1, semg1)
                start_scatter(2 * p + 1, rows1, sems1)
                wait_scatter(2 * p + 1, rows1, sems1)

        pipeline_block(0, _IB)
        pipeline_block(_IB, _NCH - _IB)

        plsc.subcore_barrier()
        pltpu.sync_copy(acc_sh.at[pl.ds(s * _WS, _WS)],
                        out_hbm.at[c].at[pl.ds(s * _WS, _WS)])

        @pl.when(s == 0)
        def _():
            pltpu.sync_copy(acc_sh.at[pl.ds(_WS * _NS, _WT)],
                            out_hbm.at[c].at[pl.ds(_WS * _NS, _WT)])

    return k(h2, src, dst)


def _mm1_body(h2_ref, agg2_ref, w1_ref, b1_ref, z_ref, st_ref):
    hb = jnp.concatenate([h2_ref[0] + agg2_ref[0], h2_ref[1] + agg2_ref[1]],
                         axis=1)
    z = _dot(hb, w1_ref[...]) + b1_ref[...]
    z_ref[...] = z
    st = jnp.stack([jnp.sum(z, axis=0), jnp.sum(z * z, axis=0)])

    @pl.when(pl.program_id(0) == 0)
    def _():
        st_ref[...] = st

    @pl.when(pl.program_id(0) > 0)
    def _():
        st_ref[...] += st


def _mm1(h2, agg2, w1, b1):
    return pl.pallas_call(
        _mm1_body,
        grid=(N // _BM,),
        in_specs=[
            pl.BlockSpec((_NC, _BM, DH), lambda i: (0, i, 0)),
            pl.BlockSpec((_NC, _BM, DH), lambda i: (0, i, 0)),
            pl.BlockSpec((D, D), lambda i: (0, 0)),
            pl.BlockSpec((1, D), lambda i: (0, 0)),
        ],
        out_specs=[
            pl.BlockSpec((_BM, D), lambda i: (i, 0)),
            pl.BlockSpec((2, D), lambda i: (0, 0)),
        ],
        out_shape=[
            jax.ShapeDtypeStruct((N, D), jnp.float32),
            jax.ShapeDtypeStruct((2, D), jnp.float32),
        ],
    )(h2, agg2, w1, b1)


def _mm2_body(z_ref, st_ref, w2_ref, b2_ref, g1_ref, beta1_ref, h2_ref):
    st = st_ref[...]
    mean = st[0:1] / N
    var = st[1:2] / N - mean * mean
    a = jnp.maximum((z_ref[...] - mean) / jnp.sqrt(var + BN_EPS)
                    * g1_ref[...] + beta1_ref[...], 0.0)
    o = jnp.maximum(_dot(a, w2_ref[...]) + b2_ref[...], 0.0)
    h2_ref[0] = o[:, :DH]
    h2_ref[1] = o[:, DH:]


def _mm2(z, st, w2, b2, g1, beta1):
    return pl.pallas_call(
        _mm2_body,
        grid=(N // _BM,),
        in_specs=[
            pl.BlockSpec((_BM, D), lambda i: (i, 0)),
            pl.BlockSpec((2, D), lambda i: (0, 0)),
            pl.BlockSpec((D, D), lambda i: (0, 0)),
            pl.BlockSpec((1, D), lambda i: (0, 0)),
            pl.BlockSpec((1, D), lambda i: (0, 0)),
            pl.BlockSpec((1, D), lambda i: (0, 0)),
        ],
        out_specs=pl.BlockSpec((_NC, _BM, DH), lambda i: (0, i, 0)),
        out_shape=jax.ShapeDtypeStruct((_NC, N, DH), jnp.float32),
    )(z, st, w2, b2, g1, beta1)


def _pool_body(h2_ref, b3_ref, sums_ref, cnt_ref):
    hb = jnp.concatenate([h2_ref[0], h2_ref[1]], axis=1)
    row = b3_ref[0].astype(jnp.int32)
    seg = lax.broadcasted_iota(jnp.int32, (G, _BM), 0)
    oh = (row == seg).astype(jnp.float32)
    ps = lax.dot_general(oh, hb, (((1,), (0,)), ((), ())),
                         preferred_element_type=jnp.float32,
                         precision=lax.Precision.HIGHEST)
    pc = jnp.broadcast_to(jnp.sum(oh, axis=1, keepdims=True), (G, DH))

    @pl.when(pl.program_id(0) == 0)
    def _():
        sums_ref[...] = ps
        cnt_ref[...] = pc

    @pl.when(pl.program_id(0) > 0)
    def _():
        sums_ref[...] += ps
        cnt_ref[...] += pc


def _pool(h2, batch3):
    return pl.pallas_call(
        _pool_body,
        grid=(N // _BM,),
        in_specs=[
            pl.BlockSpec((_NC, _BM, DH), lambda i: (0, i, 0)),
            pl.BlockSpec((1, 1, _BM), lambda i: (i, 0, 0)),
        ],
        out_specs=[
            pl.BlockSpec((G, D), lambda i: (0, 0)),
            pl.BlockSpec((G, DH), lambda i: (0, 0)),
        ],
        out_shape=[
            jax.ShapeDtypeStruct((G, D), jnp.float32),
            jax.ShapeDtypeStruct((G, DH), jnp.float32),
        ],
    )(h2, batch3)


def _final_body(sums_ref, cnt_ref, w1_ref, b1_ref, g1_ref, beta1_ref,
                w2_ref, b2_ref, out_ref):
    cnt = cnt_ref[...][:, 0:1]
    pooled = sums_ref[...] / jnp.maximum(cnt, 1.0)
    u = _dot(pooled, w1_ref[...]) + b1_ref[...]
    mean = jnp.mean(u, axis=0, keepdims=True)
    var = jnp.mean(u * u, axis=0, keepdims=True) - mean * mean
    un = (u - mean) / jnp.sqrt(var + BN_EPS) * g1_ref[...] + beta1_ref[...]
    out_ref[...] = _dot(jnp.maximum(un, 0.0), w2_ref[...]) + b2_ref[...]


def _final(sums, cnt, w1, b1, g1, beta1, w2, b2):
    return pl.pallas_call(
        _final_body,
        out_shape=jax.ShapeDtypeStruct((G, D), jnp.float32),
    )(sums, cnt, w1, b1, g1, beta1, w2, b2)


def kernel(x, edge_index, edge_attr, batch,
           conv0_w1, conv0_b1, conv0_g1, conv0_beta1, conv0_w2, conv0_b2,
           conv1_w1, conv1_b1, conv1_g1, conv1_beta1, conv1_w2, conv1_b2,
           final_w1, final_b1, final_g1, final_beta1, final_w2, final_b2):
    src = edge_index[0].astype(jnp.int32).reshape(_NS, _NCH, _CH)
    dst = edge_index[1].astype(jnp.int32).reshape(_NS, _NCH, _CH)
    batch3 = batch.astype(jnp.int32).reshape(N // _BM, 1, _BM)

    r2 = lambda v: v.reshape(1, D)
    h2 = jnp.stack([x[:, :DH], x[:, DH:]])

    for (w1, b1, g1, beta1, w2, b2) in (
        (conv0_w1, conv0_b1, conv0_g1, conv0_beta1, conv0_w2, conv0_b2),
        (conv1_w1, conv1_b1, conv1_g1, conv1_beta1, conv1_w2, conv1_b2),
    ):
        agg2 = _sc_agg(h2, src, dst)
        z, st = _mm1(h2, agg2, w1, r2(b1))
        h2 = _mm2(z, st, w2, r2(b2), r2(g1), r2(beta1))

    sums, cnt = _pool(h2, batch3)
    return _final(sums, cnt, final_w1, r2(final_b1), r2(final_g1),
                  r2(final_beta1), final_w2, r2(final_b2))


# speedup vs baseline: 1.0685x; 1.0685x over previous
import functools

import jax
import jax.numpy as jnp
from jax import lax
from jax.experimental import pallas as pl
from jax.experimental.pallas import tpu as pltpu
from jax.experimental.pallas import tpu_sc as plsc

N = 10000
D = 256
DH = 128
E = 160000
G = 64
BN_EPS = 1e-5

_NC = 2
_NS = 16
_EC = E // _NS
_CH = 80
_IB = 64
_NCH = _EC // _CH
_WS = 624
_WT = N - _WS * _NS
_ZR = 16

_PREC = lax.Precision.DEFAULT
_BM = 1000


def _dot(a, b):
    return lax.dot_general(a, b, (((1,), (0,)), ((), ())),
                           preferred_element_type=jnp.float32,
                           precision=_PREC)


@jax.jit
def _sc_agg(h2, src, dst):
    mesh = plsc.VectorSubcoreMesh(core_axis_name="c", subcore_axis_name="s",
                                  num_cores=_NC, num_subcores=_NS)

    @functools.partial(
        pl.kernel,
        out_type=jax.ShapeDtypeStruct((_NC, N, DH), jnp.float32),
        mesh=mesh,
        scratch_types=[
            pltpu.VMEM_SHARED((N, DH), jnp.float32),
            pltpu.VMEM((_ZR, DH), jnp.float32),
            pltpu.VMEM((_IB, _CH), jnp.int32),
            pltpu.VMEM((_IB, _CH), jnp.int32),
            pltpu.VMEM((_CH, DH), jnp.float32),
            pltpu.VMEM((_CH, DH), jnp.float32),
            pltpu.SemaphoreType.DMA,
            pltpu.SemaphoreType.DMA,
            pltpu.SemaphoreType.DMA,
            pltpu.SemaphoreType.DMA,
        ],
    )
    def k(h2_hbm, src_hbm, dst_hbm, out_hbm, acc_sh, zbuf, src_v, dst_v,
          rows0, rows1, semg0, semg1, sems0, sems1):
        c = lax.axis_index("c")
        s = lax.axis_index("s")

        zv = jnp.zeros((16,), jnp.float32)

        @pl.loop(0, _ZR)
        def _(r):
            @pl.loop(0, DH, step=16)
            def _(cc):
                zbuf.at[pl.ds(r, 1), pl.ds(cc, 16)][...] = zv.reshape(1, 16)

        @pl.loop(0, _WS, step=_ZR)
        def _(r0):
            pltpu.sync_copy(zbuf, acc_sh.at[pl.ds(s * _WS + r0, _ZR)])

        @pl.when(s == 0)
        def _():
            pltpu.sync_copy(zbuf, acc_sh.at[pl.ds(_WS * _NS, _WT)])

        plsc.subcore_barrier()

        def start_gather(j, rows, semg):
            pltpu.async_copy(h2_hbm.at[c].at[src_v.at[j]], rows, semg)

        def wait_gather(j, rows, semg):
            pltpu.make_async_copy(h2_hbm.at[c].at[src_v.at[j]], rows,
                                  semg).wait()

        def start_scatter(j, rows, sems):
            pltpu.async_copy(rows, acc_sh.at[dst_v.at[j]], sems, add=True)

        def wait_scatter(j, rows, sems):
            pltpu.make_async_copy(rows, acc_sh.at[dst_v.at[j]], sems).wait()

        def pipeline_block(off, n):
            pltpu.sync_copy(src_hbm.at[s].at[pl.ds(off, n)],
                            src_v.at[pl.ds(0, n)])
            pltpu.sync_copy(dst_hbm.at[s].at[pl.ds(off, n)],
                            dst_v.at[pl.ds(0, n)])
            start_gather(0, rows0, semg0)
            p = (n - 1) // 2

            @pl.loop(0, p)
            def _(jj):
                base = 2 * jj
                @pl.when(jj > 0)
                def _():
                    wait_scatter(base - 1, rows1, sems1)

                start_gather(base + 1, rows1, semg1)
                wait_gather(base, rows0, semg0)
                start_scatter(base, rows0, sems0)
                wait_scatter(base, rows0, sems0)
                start_gather(base + 2, rows0, semg0)
                wait_gather(base + 1, rows1, semg1)
                start_scatter(base + 1, rows1, sems1)

            if p > 0:
                wait_scatter(2 * p - 1, rows1, sems1)
            wait_gather(2 * p, rows0, semg0)
            if n - 2 * p == 2:
                start_gather(2 * p + 1, rows1, semg1)
            start_scatter(2 * p, rows0, sems0)
            wait_scatter(2 * p, rows0, sems0)
            if n - 2 * p == 2:
                wait_gather(2 * p + 1, rows1, semg1)
                start_scatter(2 * p + 1, rows1, sems1)
                wait_scatter(2 * p + 1, rows1, sems1)

        pipeline_block(0, _IB)
        pipeline_block(_IB, _NCH - _IB)

        plsc.subcore_barrier()
        pltpu.sync_copy(acc_sh.at[pl.ds(s * _WS, _WS)],
                        out_hbm.at[c].at[pl.ds(s * _WS, _WS)])

        @pl.when(s == 0)
        def _():
            pltpu.sync_copy(acc_sh.at[pl.ds(_WS * _NS, _WT)],
                            out_hbm.at[c].at[pl.ds(_WS * _NS, _WT)])

    return k(h2, src, dst)


def _mm1_phase(h2_ref, agg2_ref, w1_ref, b1_ref, z_s, st_s, m):
    hb = jnp.concatenate([h2_ref[0] + agg2_ref[0], h2_ref[1] + agg2_ref[1]],
                         axis=1)
    z = _dot(hb, w1_ref[...]) + b1_ref[...]
    z_s[pl.ds(m * _BM, _BM), :] = z
    st = jnp.stack([jnp.sum(z, axis=0), jnp.sum(z * z, axis=0)])

    @pl.when(m == 0)
    def _():
        st_s[...] = st

    @pl.when(m > 0)
    def _():
        st_s[...] += st


def _mm2_phase(w2_ref, b2_ref, g1_ref, beta1_ref, z_s, st_s, m):
    st = st_s[...]
    mean = st[0:1] / N
    var = st[1:2] / N - mean * mean
    z = z_s[pl.ds(m * _BM, _BM), :]
    a = jnp.maximum((z - mean) / jnp.sqrt(var + BN_EPS)
                    * g1_ref[...] + beta1_ref[...], 0.0)
    return jnp.maximum(_dot(a, w2_ref[...]) + b2_ref[...], 0.0)


def _layer0_body(h2_ref, agg2_ref, w1_ref, b1_ref, g1_ref, beta1_ref,
                 w2_ref, b2_ref, h2o_ref, z_s, st_s):
    ph, m = pl.program_id(0), pl.program_id(1)

    @pl.when(ph == 0)
    def _():
        _mm1_phase(h2_ref, agg2_ref, w1_ref, b1_ref, z_s, st_s, m)

    @pl.when(ph == 1)
    def _():
        o = _mm2_phase(w2_ref, b2_ref, g1_ref, beta1_ref, z_s, st_s, m)
        h2o_ref[0] = o[:, :DH]
        h2o_ref[1] = o[:, DH:]


def _layer0(h2, agg2, w1, b1, g1, beta1, w2, b2):
    nb = N // _BM
    return pl.pallas_call(
        _layer0_body,
        grid=(2, nb),
        in_specs=[
            pl.BlockSpec((_NC, _BM, DH), lambda ph, m: (0, m * (1 - ph), 0)),
            pl.BlockSpec((_NC, _BM, DH), lambda ph, m: (0, m * (1 - ph), 0)),
            pl.BlockSpec((D, D), lambda ph, m: (0, 0)),
            pl.BlockSpec((1, D), lambda ph, m: (0, 0)),
            pl.BlockSpec((1, D), lambda ph, m: (0, 0)),
            pl.BlockSpec((1, D), lambda ph, m: (0, 0)),
            pl.BlockSpec((D, D), lambda ph, m: (0, 0)),
            pl.BlockSpec((1, D), lambda ph, m: (0, 0)),
        ],
        out_specs=pl.BlockSpec((_NC, _BM, DH), lambda ph, m: (0, m * ph, 0)),
        out_shape=jax.ShapeDtypeStruct((_NC, N, DH), jnp.float32),
        scratch_shapes=[
            pltpu.VMEM((N, D), jnp.float32),
            pltpu.VMEM((2, D), jnp.float32),
        ],
    )(h2, agg2, w1, b1, g1, beta1, w2, b2)


def _layer1_pool_body(h2_ref, agg2_ref, b3_ref, w1_ref, b1_ref, g1_ref,
                      beta1_ref, w2_ref, b2_ref, sums_ref, cnt_ref, z_s, st_s):
    ph, m = pl.program_id(0), pl.program_id(1)

    @pl.when(ph == 0)
    def _():
        _mm1_phase(h2_ref, agg2_ref, w1_ref, b1_ref, z_s, st_s, m)

    @pl.when(ph == 1)
    def _():
        o = _mm2_phase(w2_ref, b2_ref, g1_ref, beta1_ref, z_s, st_s, m)
        row = b3_ref[0].astype(jnp.int32)
        seg = lax.broadcasted_iota(jnp.int32, (G, _BM), 0)
        oh = (row == seg).astype(jnp.float32)
        ps = lax.dot_general(oh, o, (((1,), (0,)), ((), ())),
                             preferred_element_type=jnp.float32,
                             precision=lax.Precision.HIGHEST)
        pc = jnp.broadcast_to(jnp.sum(oh, axis=1, keepdims=True), (G, DH))

        @pl.when(m == 0)
        def _():
            sums_ref[...] = ps
            cnt_ref[...] = pc

        @pl.when(m > 0)
        def _():
            sums_ref[...] += ps
            cnt_ref[...] += pc


def _layer1_pool(h2, agg2, batch3, w1, b1, g1, beta1, w2, b2):
    nb = N // _BM
    return pl.pallas_call(
        _layer1_pool_body,
        grid=(2, nb),
        in_specs=[
            pl.BlockSpec((_NC, _BM, DH), lambda ph, m: (0, m * (1 - ph), 0)),
            pl.BlockSpec((_NC, _BM, DH), lambda ph, m: (0, m * (1 - ph), 0)),
            pl.BlockSpec((1, 1, _BM), lambda ph, m: (m * ph, 0, 0)),
            pl.BlockSpec((D, D), lambda ph, m: (0, 0)),
            pl.BlockSpec((1, D), lambda ph, m: (0, 0)),
            pl.BlockSpec((1, D), lambda ph, m: (0, 0)),
            pl.BlockSpec((1, D), lambda ph, m: (0, 0)),
            pl.BlockSpec((D, D), lambda ph, m: (0, 0)),
            pl.BlockSpec((1, D), lambda ph, m: (0, 0)),
        ],
        out_specs=[
            pl.BlockSpec((G, D), lambda ph, m: (0, 0)),
            pl.BlockSpec((G, DH), lambda ph, m: (0, 0)),
        ],
        out_shape=[
            jax.ShapeDtypeStruct((G, D), jnp.float32),
            jax.ShapeDtypeStruct((G, DH), jnp.float32),
        ],
        scratch_shapes=[
            pltpu.VMEM((N, D), jnp.float32),
            pltpu.VMEM((2, D), jnp.float32),
        ],
    )(h2, agg2, batch3, w1, b1, g1, beta1, w2, b2)


def _final_body(sums_ref, cnt_ref, w1_ref, b1_ref, g1_ref, beta1_ref,
                w2_ref, b2_ref, out_ref):
    cnt = cnt_ref[...][:, 0:1]
    pooled = sums_ref[...] / jnp.maximum(cnt, 1.0)
    u = _dot(pooled, w1_ref[...]) + b1_ref[...]
    mean = jnp.mean(u, axis=0, keepdims=True)
    var = jnp.mean(u * u, axis=0, keepdims=True) - mean * mean
    un = (u - mean) / jnp.sqrt(var + BN_EPS) * g1_ref[...] + beta1_ref[...]
    out_ref[...] = _dot(jnp.maximum(un, 0.0), w2_ref[...]) + b2_ref[...]


def _final(sums, cnt, w1, b1, g1, beta1, w2, b2):
    return pl.pallas_call(
        _final_body,
        out_shape=jax.ShapeDtypeStruct((G, D), jnp.float32),
    )(sums, cnt, w1, b1, g1, beta1, w2, b2)


def kernel(x, edge_index, edge_attr, batch,
           conv0_w1, conv0_b1, conv0_g1, conv0_beta1, conv0_w2, conv0_b2,
           conv1_w1, conv1_b1, conv1_g1, conv1_beta1, conv1_w2, conv1_b2,
           final_w1, final_b1, final_g1, final_beta1, final_w2, final_b2):
    src = edge_index[0].astype(jnp.int32).reshape(_NS, _NCH, _CH)
    dst = edge_index[1].astype(jnp.int32).reshape(_NS, _NCH, _CH)
    batch3 = batch.astype(jnp.int32).reshape(N // _BM, 1, _BM)

    r2 = lambda v: v.reshape(1, D)
    x2 = jnp.stack([x[:, :DH], x[:, DH:]])

    agg0 = _sc_agg(x2, src, dst)
    h2 = _layer0(x2, agg0, conv0_w1, r2(conv0_b1), r2(conv0_g1),
                 r2(conv0_beta1), conv0_w2, r2(conv0_b2))
    agg1 = _sc_agg(h2, src, dst)
    sums, cnt = _layer1_pool(h2, agg1, batch3, conv1_w1, r2(conv1_b1),
                             r2(conv1_g1), r2(conv1_beta1), conv1_w2,
                             r2(conv1_b2))
    return _final(sums, cnt, final_w1, r2(final_b1), r2(final_g1),
                  r2(final_beta1), final_w2, r2(final_b2))


# speedup vs baseline: 1.0816x; 1.0123x over previous
import functools

import jax
import jax.numpy as jnp
from jax import lax
from jax.experimental import pallas as pl
from jax.experimental.pallas import tpu as pltpu
from jax.experimental.pallas import tpu_sc as plsc

N = 10000
D = 256
DH = 128
E = 160000
G = 64
BN_EPS = 1e-5

_NC = 2
_NS = 16
_EC = E // _NS
_CH = 80
_IB = 64
_NCH = _EC // _CH
_WS = 624
_WT = N - _WS * _NS
_ZR = 16

_PREC = lax.Precision.DEFAULT
_BM = 1000


def _dot(a, b):
    return lax.dot_general(a, b, (((1,), (0,)), ((), ())),
                           preferred_element_type=jnp.float32,
                           precision=_PREC)


@jax.jit
def _sc_agg(h2, src, dst):
    mesh = plsc.VectorSubcoreMesh(core_axis_name="c", subcore_axis_name="s",
                                  num_cores=_NC, num_subcores=_NS)

    @functools.partial(
        pl.kernel,
        out_type=jax.ShapeDtypeStruct((_NC, N, DH), jnp.float32),
        mesh=mesh,
        scratch_types=[
            pltpu.VMEM_SHARED((N, DH), jnp.float32),
            pltpu.VMEM((_ZR, DH), jnp.float32),
            pltpu.VMEM((_IB, _CH), jnp.int32),
            pltpu.VMEM((_IB, _CH), jnp.int32),
            pltpu.VMEM((_CH, DH), jnp.float32),
            pltpu.VMEM((_CH, DH), jnp.float32),
            pltpu.SemaphoreType.DMA,
            pltpu.SemaphoreType.DMA,
            pltpu.SemaphoreType.DMA,
            pltpu.SemaphoreType.DMA,
        ],
    )
    def k(h2_hbm, src_hbm, dst_hbm, out_hbm, acc_sh, zbuf, src_v, dst_v,
          rows0, rows1, semg0, semg1, sems0, sems1):
        c = lax.axis_index("c")
        s = lax.axis_index("s")

        zv = jnp.zeros((16,), jnp.float32)

        @pl.loop(0, _ZR)
        def _(r):
            @pl.loop(0, DH, step=16)
            def _(cc):
                zbuf.at[pl.ds(r, 1), pl.ds(cc, 16)][...] = zv.reshape(1, 16)

        @pl.loop(0, _WS, step=_ZR)
        def _(r0):
            pltpu.async_copy(zbuf, acc_sh.at[pl.ds(s * _WS + r0, _ZR)], sems0)

        @pl.when(s == 0)
        def _():
            pltpu.async_copy(zbuf, acc_sh.at[pl.ds(_WS * _NS, _WT)],
                             sems1)

        @pl.loop(0, _WS, step=_ZR)
        def _(r0):
            pltpu.make_async_copy(zbuf, acc_sh.at[pl.ds(s * _WS + r0, _ZR)],
                                  sems0).wait()

        @pl.when(s == 0)
        def _():
            pltpu.make_async_copy(zbuf, acc_sh.at[pl.ds(_WS * _NS, _WT)],
                                  sems1).wait()

        plsc.subcore_barrier()

        def start_gather(j, rows, semg):
            pltpu.async_copy(h2_hbm.at[c].at[src_v.at[j]], rows, semg)

        def wait_gather(j, rows, semg):
            pltpu.make_async_copy(h2_hbm.at[c].at[src_v.at[j]], rows,
                                  semg).wait()

        def start_scatter(j, rows, sems):
            pltpu.async_copy(rows, acc_sh.at[dst_v.at[j]], sems, add=True)

        def wait_scatter(j, rows, sems):
            pltpu.make_async_copy(rows, acc_sh.at[dst_v.at[j]], sems).wait()

        def pipeline_block(off, n):
            pltpu.sync_copy(src_hbm.at[s].at[pl.ds(off, n)],
                            src_v.at[pl.ds(0, n)])
            pltpu.sync_copy(dst_hbm.at[s].at[pl.ds(off, n)],
                            dst_v.at[pl.ds(0, n)])
            start_gather(0, rows0, semg0)
            p = (n - 1) // 2

            @pl.loop(0, p)
            def _(jj):
                base = 2 * jj
                @pl.when(jj > 0)
                def _():
                    wait_scatter(base - 1, rows1, sems1)

                start_gather(base + 1, rows1, semg1)
                wait_gather(base, rows0, semg0)
                start_scatter(base, rows0, sems0)
                wait_scatter(base, rows0, sems0)
                start_gather(base + 2, rows0, semg0)
                wait_gather(base + 1, rows1, semg1)
                start_scatter(base + 1, rows1, sems1)

            if p > 0:
                wait_scatter(2 * p - 1, rows1, sems1)
            wait_gather(2 * p, rows0, semg0)
            if n - 2 * p == 2:
                start_gather(2 * p + 1, rows1, semg1)
            start_scatter(2 * p, rows0, sems0)
            wait_scatter(2 * p, rows0, sems0)
            if n - 2 * p == 2:
                wait_gather(2 * p + 1, rows1, semg1)
                start_scatter(2 * p + 1, rows1, sems1)
                wait_scatter(2 * p + 1, rows1, sems1)

        pipeline_block(0, _IB)
        pipeline_block(_IB, _NCH - _IB)

        plsc.subcore_barrier()
        pltpu.sync_copy(acc_sh.at[pl.ds(s * _WS, _WS)],
                        out_hbm.at[c].at[pl.ds(s * _WS, _WS)])

        @pl.when(s == 0)
        def _():
            pltpu.sync_copy(acc_sh.at[pl.ds(_WS * _NS, _WT)],
                            out_hbm.at[c].at[pl.ds(_WS * _NS, _WT)])

    return k(h2, src, dst)


def _mm1_phase(h2_ref, agg2_ref, w1_ref, b1_ref, z_s, st_s, m):
    hb = jnp.concatenate([h2_ref[0] + agg2_ref[0], h2_ref[1] + agg2_ref[1]],
                         axis=1)
    z = _dot(hb, w1_ref[...]) + b1_ref[...]
    z_s[pl.ds(m * _BM, _BM), :] = z
    st = jnp.stack([jnp.sum(z, axis=0), jnp.sum(z * z, axis=0)])

    @pl.when(m == 0)
    def _():
        st_s[...] = st

    @pl.when(m > 0)
    def _():
        st_s[...] += st


def _mm2_phase(w2_ref, b2_ref, g1_ref, beta1_ref, z_s, st_s, m):
    st = st_s[...]
    mean = st[0:1] / N
    var = st[1:2] / N - mean * mean
    z = z_s[pl.ds(m * _BM, _BM), :]
    a = jnp.maximum((z - mean) / jnp.sqrt(var + BN_EPS)
                    * g1_ref[...] + beta1_ref[...], 0.0)
    return jnp.maximum(_dot(a, w2_ref[...]) + b2_ref[...], 0.0)


def _layer0_body(h2_ref, agg2_ref, w1_ref, b1_ref, g1_ref, beta1_ref,
                 w2_ref, b2_ref, h2o_ref, z_s, st_s):
    ph, m = pl.program_id(0), pl.program_id(1)

    @pl.when(ph == 0)
    def _():
        _mm1_phase(h2_ref, agg2_ref, w1_ref, b1_ref, z_s, st_s, m)

    @pl.when(ph == 1)
    def _():
        o = _mm2_phase(w2_ref, b2_ref, g1_ref, beta1_ref, z_s, st_s, m)
        h2o_ref[0] = o[:, :DH]
        h2o_ref[1] = o[:, DH:]


def _layer0(h2, agg2, w1, b1, g1, beta1, w2, b2):
    nb = N // _BM
    return pl.pallas_call(
        _layer0_body,
        grid=(2, nb),
        in_specs=[
            pl.BlockSpec((_NC, _BM, DH), lambda ph, m: (0, m * (1 - ph), 0)),
            pl.BlockSpec((_NC, _BM, DH), lambda ph, m: (0, m * (1 - ph), 0)),
            pl.BlockSpec((D, D), lambda ph, m: (0, 0)),
            pl.BlockSpec((1, D), lambda ph, m: (0, 0)),
            pl.BlockSpec((1, D), lambda ph, m: (0, 0)),
            pl.BlockSpec((1, D), lambda ph, m: (0, 0)),
            pl.BlockSpec((D, D), lambda ph, m: (0, 0)),
            pl.BlockSpec((1, D), lambda ph, m: (0, 0)),
        ],
        out_specs=pl.BlockSpec((_NC, _BM, DH), lambda ph, m: (0, m * ph, 0)),
        out_shape=jax.ShapeDtypeStruct((_NC, N, DH), jnp.float32),
        scratch_shapes=[
            pltpu.VMEM((N, D), jnp.float32),
            pltpu.VMEM((2, D), jnp.float32),
        ],
    )(h2, agg2, w1, b1, g1, beta1, w2, b2)


def _layer1_pool_body(h2_ref, agg2_ref, b3_ref, w1_ref, b1_ref, g1_ref,
                      beta1_ref, w2_ref, b2_ref, sums_ref, cnt_ref, z_s, st_s):
    ph, m = pl.program_id(0), pl.program_id(1)

    @pl.when(ph == 0)
    def _():
        _mm1_phase(h2_ref, agg2_ref, w1_ref, b1_ref, z_s, st_s, m)

    @pl.when(ph == 1)
    def _():
        o = _mm2_phase(w2_ref, b2_ref, g1_ref, beta1_ref, z_s, st_s, m)
        row = b3_ref[0].astype(jnp.int32)
        seg = lax.broadcasted_iota(jnp.int32, (G, _BM), 0)
        oh = (row == seg).astype(jnp.float32)
        ps = lax.dot_general(oh, o, (((1,), (0,)), ((), ())),
                             preferred_element_type=jnp.float32,
                             precision=lax.Precision.HIGHEST)
        pc = jnp.broadcast_to(jnp.sum(oh, axis=1, keepdims=True), (G, DH))

        @pl.when(m == 0)
        def _():
            sums_ref[...] = ps
            cnt_ref[...] = pc

        @pl.when(m > 0)
        def _():
            sums_ref[...] += ps
            cnt_ref[...] += pc


def _layer1_pool(h2, agg2, batch3, w1, b1, g1, beta1, w2, b2):
    nb = N // _BM
    return pl.pallas_call(
        _layer1_pool_body,
        grid=(2, nb),
        in_specs=[
            pl.BlockSpec((_NC, _BM, DH), lambda ph, m: (0, m * (1 - ph), 0)),
            pl.BlockSpec((_NC, _BM, DH), lambda ph, m: (0, m * (1 - ph), 0)),
            pl.BlockSpec((1, 1, _BM), lambda ph, m: (m * ph, 0, 0)),
            pl.BlockSpec((D, D), lambda ph, m: (0, 0)),
            pl.BlockSpec((1, D), lambda ph, m: (0, 0)),
            pl.BlockSpec((1, D), lambda ph, m: (0, 0)),
            pl.BlockSpec((1, D), lambda ph, m: (0, 0)),
            pl.BlockSpec((D, D), lambda ph, m: (0, 0)),
            pl.BlockSpec((1, D), lambda ph, m: (0, 0)),
        ],
        out_specs=[
            pl.BlockSpec((G, D), lambda ph, m: (0, 0)),
            pl.BlockSpec((G, DH), lambda ph, m: (0, 0)),
        ],
        out_shape=[
            jax.ShapeDtypeStruct((G, D), jnp.float32),
            jax.ShapeDtypeStruct((G, DH), jnp.float32),
        ],
        scratch_shapes=[
            pltpu.VMEM((N, D), jnp.float32),
            pltpu.VMEM((2, D), jnp.float32),
        ],
    )(h2, agg2, batch3, w1, b1, g1, beta1, w2, b2)


def _final_body(sums_ref, cnt_ref, w1_ref, b1_ref, g1_ref, beta1_ref,
                w2_ref, b2_ref, out_ref):
    cnt = cnt_ref[...][:, 0:1]
    pooled = sums_ref[...] / jnp.maximum(cnt, 1.0)
    u = _dot(pooled, w1_ref[...]) + b1_ref[...]
    mean = jnp.mean(u, axis=0, keepdims=True)
    var = jnp.mean(u * u, axis=0, keepdims=True) - mean * mean
    un = (u - mean) / jnp.sqrt(var + BN_EPS) * g1_ref[...] + beta1_ref[...]
    out_ref[...] = _dot(jnp.maximum(un, 0.0), w2_ref[...]) + b2_ref[...]


def _final(sums, cnt, w1, b1, g1, beta1, w2, b2):
    return pl.pallas_call(
        _final_body,
        out_shape=jax.ShapeDtypeStruct((G, D), jnp.float32),
    )(sums, cnt, w1, b1, g1, beta1, w2, b2)


def kernel(x, edge_index, edge_attr, batch,
           conv0_w1, conv0_b1, conv0_g1, conv0_beta1, conv0_w2, conv0_b2,
           conv1_w1, conv1_b1, conv1_g1, conv1_beta1, conv1_w2, conv1_b2,
           final_w1, final_b1, final_g1, final_beta1, final_w2, final_b2):
    src = edge_index[0].astype(jnp.int32).reshape(_NS, _NCH, _CH)
    dst = edge_index[1].astype(jnp.int32).reshape(_NS, _NCH, _CH)
    batch3 = batch.astype(jnp.int32).reshape(N // _BM, 1, _BM)

    r2 = lambda v: v.reshape(1, D)
    x2 = jnp.stack([x[:, :DH], x[:, DH:]])

    agg0 = _sc_agg(x2, src, dst)
    h2 = _layer0(x2, agg0, conv0_w1, r2(conv0_b1), r2(conv0_g1),
                 r2(conv0_beta1), conv0_w2, r2(conv0_b2))
    agg1 = _sc_agg(h2, src, dst)
    sums, cnt = _layer1_pool(h2, agg1, batch3, conv1_w1, r2(conv1_b1),
                             r2(conv1_g1), r2(conv1_beta1), conv1_w2,
                             r2(conv1_b2))
    return _final(sums, cnt, final_w1, r2(final_b1), r2(final_g1),
                  r2(final_beta1), final_w2, r2(final_b2))


# speedup vs baseline: 1.1115x; 1.0277x over previous
import functools

import jax
import jax.numpy as jnp
from jax import lax
from jax.experimental import pallas as pl
from jax.experimental.pallas import tpu as pltpu
from jax.experimental.pallas import tpu_sc as plsc

N = 10000
D = 256
DH = 128
E = 160000
G = 64
BN_EPS = 1e-5

_NC = 2
_NS = 16
_EC = E // _NS
_CH = 80
_IB = 64
_NCH = _EC // _CH
_WS = 624
_WT = N - _WS * _NS
_ZR = 16

_PREC = lax.Precision.DEFAULT
_BM = 2000


def _dot(a, b):
    return lax.dot_general(a, b, (((1,), (0,)), ((), ())),
                           preferred_element_type=jnp.float32,
                           precision=_PREC)


@jax.jit
def _sc_agg(h2, src, dst):
    mesh = plsc.VectorSubcoreMesh(core_axis_name="c", subcore_axis_name="s",
                                  num_cores=_NC, num_subcores=_NS)

    @functools.partial(
        pl.kernel,
        out_type=jax.ShapeDtypeStruct((_NC, N, DH), jnp.float32),
        mesh=mesh,
        scratch_types=[
            pltpu.VMEM_SHARED((N, DH), jnp.float32),
            pltpu.VMEM((_ZR, DH), jnp.float32),
            pltpu.VMEM((_IB, _CH), jnp.int32),
            pltpu.VMEM((_IB, _CH), jnp.int32),
            pltpu.VMEM((_CH, DH), jnp.float32),
            pltpu.VMEM((_CH, DH), jnp.float32),
            pltpu.SemaphoreType.DMA,
            pltpu.SemaphoreType.DMA,
            pltpu.SemaphoreType.DMA,
            pltpu.SemaphoreType.DMA,
        ],
    )
    def k(h2_hbm, src_hbm, dst_hbm, out_hbm, acc_sh, zbuf, src_v, dst_v,
          rows0, rows1, semg0, semg1, sems0, sems1):
        c = lax.axis_index("c")
        s = lax.axis_index("s")

        zv = jnp.zeros((16,), jnp.float32)

        @pl.loop(0, _ZR)
        def _(r):
            @pl.loop(0, DH, step=16)
            def _(cc):
                zbuf.at[pl.ds(r, 1), pl.ds(cc, 16)][...] = zv.reshape(1, 16)

        @pl.loop(0, _WS, step=_ZR)
        def _(r0):
            pltpu.async_copy(zbuf, acc_sh.at[pl.ds(s * _WS + r0, _ZR)], sems0)

        @pl.when(s == 0)
        def _():
            pltpu.async_copy(zbuf, acc_sh.at[pl.ds(_WS * _NS, _WT)],
                             sems1)

        @pl.loop(0, _WS, step=_ZR)
        def _(r0):
            pltpu.make_async_copy(zbuf, acc_sh.at[pl.ds(s * _WS + r0, _ZR)],
                                  sems0).wait()

        @pl.when(s == 0)
        def _():
            pltpu.make_async_copy(zbuf, acc_sh.at[pl.ds(_WS * _NS, _WT)],
                                  sems1).wait()

        plsc.subcore_barrier()

        def start_gather(j, rows, semg):
            pltpu.async_copy(h2_hbm.at[c].at[src_v.at[j]], rows, semg)

        def wait_gather(j, rows, semg):
            pltpu.make_async_copy(h2_hbm.at[c].at[src_v.at[j]], rows,
                                  semg).wait()

        def start_scatter(j, rows, sems):
            pltpu.async_copy(rows, acc_sh.at[dst_v.at[j]], sems, add=True)

        def wait_scatter(j, rows, sems):
            pltpu.make_async_copy(rows, acc_sh.at[dst_v.at[j]], sems).wait()

        def pipeline_block(off, n):
            pltpu.sync_copy(src_hbm.at[s].at[pl.ds(off, n)],
                            src_v.at[pl.ds(0, n)])
            pltpu.sync_copy(dst_hbm.at[s].at[pl.ds(off, n)],
                            dst_v.at[pl.ds(0, n)])
            start_gather(0, rows0, semg0)
            p = (n - 1) // 2

            @pl.loop(0, p)
            def _(jj):
                base = 2 * jj
                @pl.when(jj > 0)
                def _():
                    wait_scatter(base - 1, rows1, sems1)

                start_gather(base + 1, rows1, semg1)
                wait_gather(base, rows0, semg0)
                start_scatter(base, rows0, sems0)
                wait_scatter(base, rows0, sems0)
                start_gather(base + 2, rows0, semg0)
                wait_gather(base + 1, rows1, semg1)
                start_scatter(base + 1, rows1, sems1)

            if p > 0:
                wait_scatter(2 * p - 1, rows1, sems1)
            wait_gather(2 * p, rows0, semg0)
            if n - 2 * p == 2:
                start_gather(2 * p + 1, rows1, semg1)
            start_scatter(2 * p, rows0, sems0)
            wait_scatter(2 * p, rows0, sems0)
            if n - 2 * p == 2:
                wait_gather(2 * p + 1, rows1, semg1)
                start_scatter(2 * p + 1, rows1, sems1)
                wait_scatter(2 * p + 1, rows1, sems1)

        pipeline_block(0, _IB)
        pipeline_block(_IB, _NCH - _IB)

        plsc.subcore_barrier()
        pltpu.sync_copy(acc_sh.at[pl.ds(s * _WS, _WS)],
                        out_hbm.at[c].at[pl.ds(s * _WS, _WS)])

        @pl.when(s == 0)
        def _():
            pltpu.sync_copy(acc_sh.at[pl.ds(_WS * _NS, _WT)],
                            out_hbm.at[c].at[pl.ds(_WS * _NS, _WT)])

    return k(h2, src, dst)


def _mm1_phase(h2_ref, agg2_ref, w1_ref, b1_ref, z_s, st_s, m):
    hb = jnp.concatenate([h2_ref[0] + agg2_ref[0], h2_ref[1] + agg2_ref[1]],
                         axis=1)
    z = _dot(hb, w1_ref[...]) + b1_ref[...]
    z_s[pl.ds(m * _BM, _BM), :] = z
    st = jnp.stack([jnp.sum(z, axis=0), jnp.sum(z * z, axis=0)])

    @pl.when(m == 0)
    def _():
        st_s[...] = st

    @pl.when(m > 0)
    def _():
        st_s[...] += st


def _mm2_phase(w2_ref, b2_ref, g1_ref, beta1_ref, z_s, st_s, m):
    st = st_s[...]
    mean = st[0:1] / N
    var = st[1:2] / N - mean * mean
    z = z_s[pl.ds(m * _BM, _BM), :]
    a = jnp.maximum((z - mean) / jnp.sqrt(var + BN_EPS)
                    * g1_ref[...] + beta1_ref[...], 0.0)
    return jnp.maximum(_dot(a, w2_ref[...]) + b2_ref[...], 0.0)


def _layer0_body(h2_ref, agg2_ref, w1_ref, b1_ref, g1_ref, beta1_ref,
                 w2_ref, b2_ref, h2o_ref, z_s, st_s):
    ph, m = pl.program_id(0), pl.program_id(1)

    @pl.when(ph == 0)
    def _():
        _mm1_phase(h2_ref, agg2_ref, w1_ref, b1_ref, z_s, st_s, m)

    @pl.when(ph == 1)
    def _():
        o = _mm2_phase(w2_ref, b2_ref, g1_ref, beta1_ref, z_s, st_s, m)
        h2o_ref[0] = o[:, :DH]
        h2o_ref[1] = o[:, DH:]


def _layer0(h2, agg2, w1, b1, g1, beta1, w2, b2):
    nb = N // _BM
    return pl.pallas_call(
        _layer0_body,
        grid=(2, nb),
        in_specs=[
            pl.BlockSpec((_NC, _BM, DH), lambda ph, m: (0, m * (1 - ph), 0)),
            pl.BlockSpec((_NC, _BM, DH), lambda ph, m: (0, m * (1 - ph), 0)),
            pl.BlockSpec((D, D), lambda ph, m: (0, 0)),
            pl.BlockSpec((1, D), lambda ph, m: (0, 0)),
            pl.BlockSpec((1, D), lambda ph, m: (0, 0)),
            pl.BlockSpec((1, D), lambda ph, m: (0, 0)),
            pl.BlockSpec((D, D), lambda ph, m: (0, 0)),
            pl.BlockSpec((1, D), lambda ph, m: (0, 0)),
        ],
        out_specs=pl.BlockSpec((_NC, _BM, DH), lambda ph, m: (0, m * ph, 0)),
        out_shape=jax.ShapeDtypeStruct((_NC, N, DH), jnp.float32),
        scratch_shapes=[
            pltpu.VMEM((N, D), jnp.float32),
            pltpu.VMEM((2, D), jnp.float32),
        ],
    )(h2, agg2, w1, b1, g1, beta1, w2, b2)


def _layer1_pool_body(h2_ref, agg2_ref, b3_ref, w1_ref, b1_ref, g1_ref,
                      beta1_ref, w2_ref, b2_ref, sums_ref, cnt_ref, z_s, st_s):
    ph, m = pl.program_id(0), pl.program_id(1)

    @pl.when(ph == 0)
    def _():
        _mm1_phase(h2_ref, agg2_ref, w1_ref, b1_ref, z_s, st_s, m)

    @pl.when(ph == 1)
    def _():
        o = _mm2_phase(w2_ref, b2_ref, g1_ref, beta1_ref, z_s, st_s, m)
        row = b3_ref[0].astype(jnp.int32)
        seg = lax.broadcasted_iota(jnp.int32, (G, _BM), 0)
        oh = (row == seg).astype(jnp.float32)
        ps = lax.dot_general(oh, o, (((1,), (0,)), ((), ())),
                             preferred_element_type=jnp.float32,
                             precision=lax.Precision.HIGHEST)
        pc = jnp.broadcast_to(jnp.sum(oh, axis=1, keepdims=True), (G, DH))

        @pl.when(m == 0)
        def _():
            sums_ref[...] = ps
            cnt_ref[...] = pc

        @pl.when(m > 0)
        def _():
            sums_ref[...] += ps
            cnt_ref[...] += pc


def _layer1_pool(h2, agg2, batch3, w1, b1, g1, beta1, w2, b2):
    nb = N // _BM
    return pl.pallas_call(
        _layer1_pool_body,
        grid=(2, nb),
        in_specs=[
            pl.BlockSpec((_NC, _BM, DH), lambda ph, m: (0, m * (1 - ph), 0)),
            pl.BlockSpec((_NC, _BM, DH), lambda ph, m: (0, m * (1 - ph), 0)),
            pl.BlockSpec((1, 1, _BM), lambda ph, m: (m * ph, 0, 0)),
            pl.BlockSpec((D, D), lambda ph, m: (0, 0)),
            pl.BlockSpec((1, D), lambda ph, m: (0, 0)),
            pl.BlockSpec((1, D), lambda ph, m: (0, 0)),
            pl.BlockSpec((1, D), lambda ph, m: (0, 0)),
            pl.BlockSpec((D, D), lambda ph, m: (0, 0)),
            pl.BlockSpec((1, D), lambda ph, m: (0, 0)),
        ],
        out_specs=[
            pl.BlockSpec((G, D), lambda ph, m: (0, 0)),
            pl.BlockSpec((G, DH), lambda ph, m: (0, 0)),
        ],
        out_shape=[
            jax.ShapeDtypeStruct((G, D), jnp.float32),
            jax.ShapeDtypeStruct((G, DH), jnp.float32),
        ],
        scratch_shapes=[
            pltpu.VMEM((N, D), jnp.float32),
            pltpu.VMEM((2, D), jnp.float32),
        ],
    )(h2, agg2, batch3, w1, b1, g1, beta1, w2, b2)


def _final_body(sums_ref, cnt_ref, w1_ref, b1_ref, g1_ref, beta1_ref,
                w2_ref, b2_ref, out_ref):
    cnt = cnt_ref[...][:, 0:1]
    pooled = sums_ref[...] / jnp.maximum(cnt, 1.0)
    u = _dot(pooled, w1_ref[...]) + b1_ref[...]
    mean = jnp.mean(u, axis=0, keepdims=True)
    var = jnp.mean(u * u, axis=0, keepdims=True) - mean * mean
    un = (u - mean) / jnp.sqrt(var + BN_EPS) * g1_ref[...] + beta1_ref[...]
    out_ref[...] = _dot(jnp.maximum(un, 0.0), w2_ref[...]) + b2_ref[...]


def _final(sums, cnt, w1, b1, g1, beta1, w2, b2):
    return pl.pallas_call(
        _final_body,
        out_shape=jax.ShapeDtypeStruct((G, D), jnp.float32),
    )(sums, cnt, w1, b1, g1, beta1, w2, b2)


def kernel(x, edge_index, edge_attr, batch,
           conv0_w1, conv0_b1, conv0_g1, conv0_beta1, conv0_w2, conv0_b2,
           conv1_w1, conv1_b1, conv1_g1, conv1_beta1, conv1_w2, conv1_b2,
           final_w1, final_b1, final_g1, final_beta1, final_w2, final_b2):
    src = edge_index[0].astype(jnp.int32).reshape(_NS, _NCH, _CH)
    dst = edge_index[1].astype(jnp.int32).reshape(_NS, _NCH, _CH)
    batch3 = batch.astype(jnp.int32).reshape(N // _BM, 1, _BM)

    r2 = lambda v: v.reshape(1, D)
    x2 = jnp.stack([x[:, :DH], x[:, DH:]])

    agg0 = _sc_agg(x2, src, dst)
    h2 = _layer0(x2, agg0, conv0_w1, r2(conv0_b1), r2(conv0_g1),
                 r2(conv0_beta1), conv0_w2, r2(conv0_b2))
    agg1 = _sc_agg(h2, src, dst)
    sums, cnt = _layer1_pool(h2, agg1, batch3, conv1_w1, r2(conv1_b1),
                             r2(conv1_g1), r2(conv1_beta1), conv1_w2,
                             r2(conv1_b2))
    return _final(sums, cnt, final_w1, r2(final_b1), r2(final_g1),
                  r2(final_beta1), final_w2, r2(final_b2))
